# Initial kernel scaffold; baseline (speedup 1.0000x reference)
import functools

import jax
import jax.numpy as jnp
from jax import lax
from jax.experimental import pallas as pl
from jax.experimental.pallas import tpu as pltpu
from jax.experimental.pallas import tpu_sc as plsc

N = 10000
E = 320000
DIN = 128
DH = 16
DOUT = 128

NC, NS = 2, 16
NW = NC * NS
NPAD = 10240
CH = E // NW
NBATCH = 10
B = CH // NBATCH
ROWS_PER_TILE = NPAD // NS

_mesh = plsc.VectorSubcoreMesh(core_axis_name="c", subcore_axis_name="s")


def _deg_body(dst_hbm, out_hbm, dst_v, ones_v, zrows_v, deg_sh):
    cid = lax.axis_index("c")
    sid = lax.axis_index("s")
    base = cid * (E // 2) + sid * CH

    for k in range(NBATCH):
        pltpu.sync_copy(dst_hbm.at[pl.ds(base + k * B, B)], dst_v.at[k])

    def fill_ones(i, _):
        ones_v[i, :] = jnp.ones((16,), jnp.float32)
        return 0

    lax.fori_loop(0, B, fill_ones, 0)

    def fill_zero(i, _):
        zrows_v[i, :] = jnp.zeros((16,), jnp.float32)
        return 0

    lax.fori_loop(0, ROWS_PER_TILE, fill_zero, 0)

    pltpu.sync_copy(zrows_v, deg_sh.at[pl.ds(sid * ROWS_PER_TILE, ROWS_PER_TILE)])
    plsc.subcore_barrier()

    for k in range(NBATCH):
        pltpu.sync_copy(ones_v, deg_sh.at[dst_v.at[k]], add=True)

    plsc.subcore_barrier()
    pltpu.sync_copy(
        deg_sh.at[pl.ds(sid * ROWS_PER_TILE, ROWS_PER_TILE)],
        out_hbm.at[cid, pl.ds(sid * ROWS_PER_TILE, ROWS_PER_TILE)],
    )


_deg_call = functools.partial(
    pl.kernel,
    out_type=jax.ShapeDtypeStruct((NC, NPAD, DH), jnp.float32),
    mesh=_mesh,
    scratch_types=[
        pltpu.VMEM((NBATCH, B), jnp.int32),
        pltpu.VMEM((B, DH), jnp.float32),
        pltpu.VMEM((ROWS_PER_TILE, DH), jnp.float32),
        pltpu.VMEM_SHARED((NPAD, DH), jnp.float32),
    ],
)(_deg_body)


def _agg_body(src_hbm, dst_hbm, ew_hbm, dinv_hbm, feat_hbm, out_hbm,
              src_v, dst_v, ew_v, sc_v, dinv_v, rows_v, zrows_v, agg_sh, sem):
    cid = lax.axis_index("c")
    sid = lax.axis_index("s")
    base = cid * (E // 2) + sid * CH

    pltpu.sync_copy(src_hbm.at[pl.ds(base, CH)], src_v)
    for k in range(NBATCH):
        pltpu.sync_copy(dst_hbm.at[pl.ds(base + k * B, B)], dst_v.at[k])
    pltpu.sync_copy(ew_hbm.at[pl.ds(base, CH)], ew_v)
    pltpu.sync_copy(dinv_hbm, dinv_v)

    def fill_zero(i, _):
        zrows_v[i, :] = jnp.zeros((16,), jnp.float32)
        return 0

    lax.fori_loop(0, ROWS_PER_TILE, fill_zero, 0)
    pltpu.sync_copy(zrows_v, agg_sh.at[pl.ds(sid * ROWS_PER_TILE, ROWS_PER_TILE)])

    def scl(j, _):
        s16 = src_v[pl.ds(j * 16, 16)]
        d16 = plsc.load_gather(dinv_v, [s16])
        sc_v[pl.ds(j * 16, 16)] = ew_v[pl.ds(j * 16, 16)] * d16
        return 0

    lax.fori_loop(0, CH // 16, scl, 0)
    plsc.subcore_barrier()

    for k in range(NBATCH):
        pltpu.async_copy(feat_hbm.at[src_v.at[pl.ds(k * B, B)]], rows_v, sem).wait()

        def scale_row(i, _):
            rows_v[i, :] = rows_v[i, :] * sc_v[k * B + i]
            return 0

        lax.fori_loop(0, B, scale_row, 0)
        pltpu.sync_copy(rows_v, agg_sh.at[dst_v.at[k]], add=True)

    plsc.subcore_barrier()
    pltpu.sync_copy(
        agg_sh.at[pl.ds(sid * ROWS_PER_TILE, ROWS_PER_TILE)],
        out_hbm.at[cid, pl.ds(sid * ROWS_PER_TILE, ROWS_PER_TILE)],
    )


_agg_call = functools.partial(
    pl.kernel,
    out_type=jax.ShapeDtypeStruct((NC, NPAD, DH), jnp.float32),
    mesh=_mesh,
    scratch_types=[
        pltpu.VMEM((CH,), jnp.int32),
        pltpu.VMEM((NBATCH, B), jnp.int32),
        pltpu.VMEM((CH,), jnp.float32),
        pltpu.VMEM((CH,), jnp.float32),
        pltpu.VMEM((NPAD,), jnp.float32),
        pltpu.VMEM((B, DH), jnp.float32),
        pltpu.VMEM((ROWS_PER_TILE, DH), jnp.float32),
        pltpu.VMEM_SHARED((NPAD, DH), jnp.float32),
        pltpu.SemaphoreType.DMA,
    ],
)(_agg_body)


def _dense1_body(xp_ref, w1_ref, degp_ref, hw1_ref, dinvb_ref, dinv2b_ref):
    hw1_ref[...] = jnp.dot(xp_ref[...], w1_ref[...],
                           preferred_element_type=jnp.float32)
    degb = degp_ref[0] + degp_ref[1] + 1.0
    dinvb_ref[...] = lax.rsqrt(degb)
    dinv2b_ref[...] = 1.0 / degb


def _ew_body(cbar_ref, ea_ref, wev_ref, ew_ref):
    blk = ea_ref[...]
    w = wev_ref[...]
    ew_ref[...] = jnp.sum(blk * w, axis=-1) + cbar_ref[0]


def _mid_body(aggp_ref, hw1_ref, dinvb_ref, dinv2b_ref, b1_ref, h_ref):
    agg = aggp_ref[0] + aggp_ref[1]
    h = dinvb_ref[...] * agg + dinv2b_ref[...] * hw1_ref[...] + b1_ref[...]
    h_ref[...] = jnp.maximum(h, 0.0)


def _fin_body(aggp_ref, h_ref, dinvb_ref, dinv2b_ref, w2_ref, b2_ref, out_ref):
    pre = dinvb_ref[...] * (aggp_ref[0] + aggp_ref[1]) \
        + dinv2b_ref[...] * h_ref[...]
    out_ref[...] = jnp.dot(pre, w2_ref[...],
                           preferred_element_type=jnp.float32) + b2_ref[...]


def kernel(x, edge_index, edge_attr, We, be, W1, b1, W2, b2):
    src = edge_index[0]
    dst = edge_index[1]
    dea = We.shape[0]

    wevec = jnp.mean(We, axis=1)
    cbar = jnp.mean(be).reshape(1)
    xp = jnp.pad(x, ((0, NPAD - N), (0, 0)))

    degp = _deg_call(dst)

    hw1, dinvb, dinv2b = pl.pallas_call(
        _dense1_body,
        out_shape=[
            jax.ShapeDtypeStruct((NPAD, DH), jnp.float32),
            jax.ShapeDtypeStruct((NPAD, DH), jnp.float32),
            jax.ShapeDtypeStruct((NPAD, DH), jnp.float32),
        ],
    )(xp, W1, degp)

    ER = E // 128
    RB = 100
    ea3 = edge_attr.reshape(ER, 128, dea)
    ew2d = pl.pallas_call(
        _ew_body,
        grid=(ER // RB,),
        in_specs=[
            pl.BlockSpec(memory_space=pltpu.SMEM),
            pl.BlockSpec((RB, 128, dea), lambda i: (i, 0, 0)),
            pl.BlockSpec((1, 1, dea), lambda i: (0, 0, 0)),
        ],
        out_specs=pl.BlockSpec((RB, 128), lambda i: (i, 0)),
        out_shape=jax.ShapeDtypeStruct((ER, 128), jnp.float32),
    )(cbar, ea3, wevec.reshape(1, 1, dea))
    ew = ew2d.reshape(E)

    dinv_flat = dinvb[:, 0]

    agg1 = _agg_call(src, dst, ew, dinv_flat, hw1)

    h = pl.pallas_call(
        _mid_body,
        out_shape=jax.ShapeDtypeStruct((NPAD, DH), jnp.float32),
    )(agg1, hw1, dinvb, dinv2b, b1.reshape(1, DH))

    agg2 = _agg_call(src, dst, ew, dinv_flat, h)

    out = pl.pallas_call(
        _fin_body,
        out_shape=jax.ShapeDtypeStruct((NPAD, DOUT), jnp.float32),
    )(agg2, h, dinvb, dinv2b, W2, b2.reshape(1, DOUT))

    return out[:N]


# speedup vs baseline: 30.1220x; 30.1220x over previous
import functools

import jax
import jax.numpy as jnp
from jax import lax
from jax.experimental import pallas as pl
from jax.experimental.pallas import tpu as pltpu
from jax.experimental.pallas import tpu_sc as plsc

N = 10000
E = 320000
DIN = 128
DH = 16
DOUT = 128

NC, NS = 2, 16
NW = NC * NS
NPAD = 10240
CH = E // NW
NBATCH = 5
B = CH // NBATCH
ROWS_PER_TILE = NPAD // NS

_mesh = plsc.VectorSubcoreMesh(core_axis_name="c", subcore_axis_name="s")
_sc_params = pltpu.CompilerParams(use_tc_tiling_on_sc=False, needs_layout_passes=False)


def _deg_body(dst_hbm, out_hbm, dst_v, ones_v, zrows_v, deg_sh):
    cid = lax.axis_index("c")
    sid = lax.axis_index("s")
    base = cid * (E // 2) + sid * CH

    pltpu.sync_copy(dst_hbm.at[pl.ds(base, CH)], dst_v)

    def fill_ones(i, _):
        ones_v[i, :] = jnp.ones((16,), jnp.float32)
        return 0

    lax.fori_loop(0, B, fill_ones, 0)

    def fill_zero(i, _):
        zrows_v[i, :] = jnp.zeros((16,), jnp.float32)
        return 0

    lax.fori_loop(0, ROWS_PER_TILE, fill_zero, 0)

    pltpu.sync_copy(zrows_v, deg_sh.at[pl.ds(sid * ROWS_PER_TILE, ROWS_PER_TILE)])
    plsc.subcore_barrier()

    for k in range(NBATCH):
        pltpu.sync_copy(ones_v, deg_sh.at[dst_v.at[pl.ds(k * B, B)]], add=True)

    plsc.subcore_barrier()
    pltpu.sync_copy(
        deg_sh.at[pl.ds(sid * ROWS_PER_TILE, ROWS_PER_TILE)],
        out_hbm.at[cid, pl.ds(sid * ROWS_PER_TILE, ROWS_PER_TILE)],
    )


_deg_call = functools.partial(
    pl.kernel,
    out_type=jax.ShapeDtypeStruct((NC, NPAD, DH), jnp.float32),
    mesh=_mesh,
    compiler_params=_sc_params,
    scratch_types=[
        pltpu.VMEM((CH,), jnp.int32),
        pltpu.VMEM((B, DH), jnp.float32),
        pltpu.VMEM((ROWS_PER_TILE, DH), jnp.float32),
        pltpu.VMEM_SHARED((NPAD, DH), jnp.float32),
    ],
)(_deg_body)


def _agg_body(src_hbm, dst_hbm, ew_hbm, dinv_hbm, feat_hbm, out_hbm,
              src_v, dst_v, ew_v, sc_v, dinv_v, rows_v, zrows_v, agg_sh, sem):
    cid = lax.axis_index("c")
    sid = lax.axis_index("s")
    base = cid * (E // 2) + sid * CH

    pltpu.sync_copy(src_hbm.at[pl.ds(base, CH)], src_v)
    pltpu.sync_copy(dst_hbm.at[pl.ds(base, CH)], dst_v)
    pltpu.sync_copy(ew_hbm.at[pl.ds(base, CH)], ew_v)
    pltpu.sync_copy(dinv_hbm, dinv_v)

    def fill_zero(i, _):
        zrows_v[i, :] = jnp.zeros((16,), jnp.float32)
        return 0

    lax.fori_loop(0, ROWS_PER_TILE, fill_zero, 0)
    pltpu.sync_copy(zrows_v, agg_sh.at[pl.ds(sid * ROWS_PER_TILE, ROWS_PER_TILE)])

    def scl(j, _):
        s16 = src_v[pl.ds(j * 16, 16)]
        d16 = plsc.load_gather(dinv_v, [s16])
        sc_v[pl.ds(j * 16, 16)] = ew_v[pl.ds(j * 16, 16)] * d16
        return 0

    lax.fori_loop(0, CH // 16, scl, 0)
    plsc.subcore_barrier()

    for k in range(NBATCH):
        pltpu.async_copy(feat_hbm.at[src_v.at[pl.ds(k * B, B)]], rows_v, sem).wait()

        def scale_rows16(j, _):
            s16 = sc_v[pl.ds(k * B + j * 16, 16)]
            for i in range(16):
                r = j * 16 + i
                rows_v[r, :] = rows_v[r, :] * s16[i]
            return 0

        lax.fori_loop(0, B // 16, scale_rows16, 0)
        pltpu.sync_copy(rows_v, agg_sh.at[dst_v.at[pl.ds(k * B, B)]], add=True)

    plsc.subcore_barrier()
    pltpu.sync_copy(
        agg_sh.at[pl.ds(sid * ROWS_PER_TILE, ROWS_PER_TILE)],
        out_hbm.at[cid, pl.ds(sid * ROWS_PER_TILE, ROWS_PER_TILE)],
    )


_agg_call = functools.partial(
    pl.kernel,
    out_type=jax.ShapeDtypeStruct((NC, NPAD, DH), jnp.float32),
    mesh=_mesh,
    compiler_params=_sc_params,
    scratch_types=[
        pltpu.VMEM((CH,), jnp.int32),
        pltpu.VMEM((CH,), jnp.int32),
        pltpu.VMEM((CH,), jnp.float32),
        pltpu.VMEM((CH,), jnp.float32),
        pltpu.VMEM((NPAD,), jnp.float32),
        pltpu.VMEM((B, DH), jnp.float32),
        pltpu.VMEM((ROWS_PER_TILE, DH), jnp.float32),
        pltpu.VMEM_SHARED((NPAD, DH), jnp.float32),
        pltpu.SemaphoreType.DMA,
    ],
)(_agg_body)


def _dense1_body(xp_ref, w1_ref, degp_ref, hw1_ref, dinvb_ref, dinv2b_ref):
    hw1_ref[...] = jnp.dot(xp_ref[...], w1_ref[...],
                           preferred_element_type=jnp.float32)
    degb = degp_ref[0] + degp_ref[1] + 1.0
    dinvb_ref[...] = lax.rsqrt(degb)
    dinv2b_ref[...] = 1.0 / degb


def _ew_body(cbar_ref, ea_ref, wev_ref, ew_ref):
    blk = ea_ref[...]
    w = wev_ref[...]
    ew_ref[...] = jnp.sum(blk * w, axis=-1) + cbar_ref[0]


def _mid_body(aggp_ref, hw1_ref, dinvb_ref, dinv2b_ref, b1_ref, h_ref):
    agg = aggp_ref[0] + aggp_ref[1]
    h = dinvb_ref[...] * agg + dinv2b_ref[...] * hw1_ref[...] + b1_ref[...]
    h_ref[...] = jnp.maximum(h, 0.0)


def _fin_body(aggp_ref, h_ref, dinvb_ref, dinv2b_ref, w2_ref, b2_ref, out_ref):
    pre = dinvb_ref[...] * (aggp_ref[0] + aggp_ref[1]) \
        + dinv2b_ref[...] * h_ref[...]
    out_ref[...] = jnp.dot(pre, w2_ref[...],
                           preferred_element_type=jnp.float32) + b2_ref[...]


def kernel(x, edge_index, edge_attr, We, be, W1, b1, W2, b2):
    src = edge_index[0]
    dst = edge_index[1]
    dea = We.shape[0]

    wevec = jnp.mean(We, axis=1)
    cbar = jnp.mean(be).reshape(1)
    xp = jnp.pad(x, ((0, NPAD - N), (0, 0)))

    degp = _deg_call(dst)

    hw1, dinvb, dinv2b = pl.pallas_call(
        _dense1_body,
        out_shape=[
            jax.ShapeDtypeStruct((NPAD, DH), jnp.float32),
            jax.ShapeDtypeStruct((NPAD, DH), jnp.float32),
            jax.ShapeDtypeStruct((NPAD, DH), jnp.float32),
        ],
    )(xp, W1, degp)

    EC = 1000
    ER = E // EC
    RB = 16
    ea3 = edge_attr.reshape(ER, EC, dea)
    ew2d = pl.pallas_call(
        _ew_body,
        grid=(ER // RB,),
        in_specs=[
            pl.BlockSpec(memory_space=pltpu.SMEM),
            pl.BlockSpec((RB, EC, dea), lambda i: (i, 0, 0)),
            pl.BlockSpec((1, 1, dea), lambda i: (0, 0, 0)),
        ],
        out_specs=pl.BlockSpec((RB, EC), lambda i: (i, 0)),
        out_shape=jax.ShapeDtypeStruct((ER, EC), jnp.float32),
    )(cbar, ea3, wevec.reshape(1, 1, dea))
    ew = ew2d.reshape(E)

    dinv_flat = dinvb[:, 0]

    agg1 = _agg_call(src, dst, ew, dinv_flat, hw1)

    h = pl.pallas_call(
        _mid_body,
        out_shape=jax.ShapeDtypeStruct((NPAD, DH), jnp.float32),
    )(agg1, hw1, dinvb, dinv2b, b1.reshape(1, DH))

    agg2 = _agg_call(src, dst, ew, dinv_flat, h)

    out = pl.pallas_call(
        _fin_body,
        out_shape=jax.ShapeDtypeStruct((NPAD, DOUT), jnp.float32),
    )(agg2, h, dinvb, dinv2b, W2, b2.reshape(1, DOUT))

    return out[:N]


# speedup vs baseline: 35.0650x; 1.1641x over previous
import functools

import jax
import jax.numpy as jnp
from jax import lax
from jax.experimental import pallas as pl
from jax.experimental.pallas import tpu as pltpu
from jax.experimental.pallas import tpu_sc as plsc

N = 10000
E = 320000
DIN = 128
DH = 16
DOUT = 128

NC, NS = 2, 16
NW = NC * NS
CH = E // NW
NBATCH = 5
B = CH // NBATCH
ROWS_PER_TILE = N // NS

_mesh = plsc.VectorSubcoreMesh(core_axis_name="c", subcore_axis_name="s")
_sc_params = pltpu.CompilerParams(use_tc_tiling_on_sc=False,
                                  needs_layout_passes=False)


def _deg_body(ei_hbm, out_hbm, dst_v, ones_v, deg_sh):
    cid = lax.axis_index("c")
    sid = lax.axis_index("s")
    base = cid * (E // 2) + sid * CH

    pltpu.sync_copy(ei_hbm.at[1, pl.ds(base, CH)], dst_v)

    def fill_zero(i, _):
        ones_v[i, :] = jnp.zeros((16,), jnp.float32)
        return 0

    lax.fori_loop(0, ROWS_PER_TILE, fill_zero, 0)
    pltpu.sync_copy(ones_v.at[pl.ds(0, ROWS_PER_TILE)],
                    deg_sh.at[pl.ds(sid * ROWS_PER_TILE, ROWS_PER_TILE)])

    def fill_ones(i, _):
        ones_v[i, :] = jnp.ones((16,), jnp.float32)
        return 0

    lax.fori_loop(0, B, fill_ones, 0)
    plsc.subcore_barrier()

    for k in range(NBATCH):
        pltpu.sync_copy(ones_v, deg_sh.at[dst_v.at[pl.ds(k * B, B)]], add=True)

    plsc.subcore_barrier()
    pltpu.sync_copy(
        deg_sh.at[pl.ds(sid * ROWS_PER_TILE, ROWS_PER_TILE)],
        out_hbm.at[cid, pl.ds(sid * ROWS_PER_TILE, ROWS_PER_TILE)],
    )


_deg_call = functools.partial(
    pl.kernel,
    out_type=jax.ShapeDtypeStruct((NC, N, DH), jnp.float32),
    mesh=_mesh,
    compiler_params=_sc_params,
    scratch_types=[
        pltpu.VMEM((CH,), jnp.int32),
        pltpu.VMEM((B, DH), jnp.float32),
        pltpu.VMEM_SHARED((N, DH), jnp.float32),
    ],
)(_deg_body)


def _agg_body(ei_hbm, ew_hbm, dinv_hbm, feat_hbm, out_hbm,
              src_v, dst_v, ew_v, sc_v, dinv_v, rows_v0, rows_v1, agg_sh,
              gsem, ssem):
    cid = lax.axis_index("c")
    sid = lax.axis_index("s")
    base = cid * (E // 2) + sid * CH

    pltpu.sync_copy(ei_hbm.at[0, pl.ds(base, CH)], src_v)
    pltpu.sync_copy(ei_hbm.at[1, pl.ds(base, CH)], dst_v)
    pltpu.sync_copy(ew_hbm.at[pl.ds(base, CH)], ew_v)
    pltpu.sync_copy(dinv_hbm, dinv_v)

    def fill_zero(i, _):
        rows_v0[i, :] = jnp.zeros((16,), jnp.float32)
        return 0

    lax.fori_loop(0, ROWS_PER_TILE, fill_zero, 0)
    pltpu.sync_copy(rows_v0.at[pl.ds(0, ROWS_PER_TILE)],
                    agg_sh.at[pl.ds(sid * ROWS_PER_TILE, ROWS_PER_TILE)])

    def scl(j, _):
        s16 = src_v[pl.ds(j * 16, 16)]
        d16 = plsc.load_gather(dinv_v, [s16])
        sc_v[pl.ds(j * 16, 16)] = ew_v[pl.ds(j * 16, 16)] * d16
        return 0

    lax.fori_loop(0, CH // 16, scl, 0)
    plsc.subcore_barrier()

    bufs = [rows_v0, rows_v1]
    gd = [None] * NBATCH
    sd = [None] * NBATCH
    gd[0] = pltpu.async_copy(feat_hbm.at[src_v.at[pl.ds(0, B)]], bufs[0], gsem)
    for k in range(NBATCH):
        buf = bufs[k % 2]
        gd[k].wait()
        if k + 1 < NBATCH:
            if k >= 1:
                sd[k - 1].wait()
            gd[k + 1] = pltpu.async_copy(
                feat_hbm.at[src_v.at[pl.ds((k + 1) * B, B)]],
                bufs[(k + 1) % 2], gsem)

        @plsc.parallel_loop(0, B // 16)
        def _(j):
            s16 = sc_v[pl.ds(k * B + j * 16, 16)]
            for i in range(16):
                r = j * 16 + i
                buf[r, :] = buf[r, :] * s16[i]

        sd[k] = pltpu.async_copy(buf, agg_sh.at[dst_v.at[pl.ds(k * B, B)]],
                                 ssem, add=True)
    sd[NBATCH - 2].wait()
    sd[NBATCH - 1].wait()

    plsc.subcore_barrier()
    pltpu.sync_copy(
        agg_sh.at[pl.ds(sid * ROWS_PER_TILE, ROWS_PER_TILE)],
        out_hbm.at[cid, pl.ds(sid * ROWS_PER_TILE, ROWS_PER_TILE)],
    )


_agg_call = functools.partial(
    pl.kernel,
    out_type=jax.ShapeDtypeStruct((NC, N, DH), jnp.float32),
    mesh=_mesh,
    compiler_params=_sc_params,
    scratch_types=[
        pltpu.VMEM((CH,), jnp.int32),
        pltpu.VMEM((CH,), jnp.int32),
        pltpu.VMEM((CH,), jnp.float32),
        pltpu.VMEM((CH,), jnp.float32),
        pltpu.VMEM((N,), jnp.float32),
        pltpu.VMEM((B, DH), jnp.float32),
        pltpu.VMEM((B, DH), jnp.float32),
        pltpu.VMEM_SHARED((N, DH), jnp.float32),
        pltpu.SemaphoreType.DMA,
        pltpu.SemaphoreType.DMA,
    ],
)(_agg_body)


def _dense1_body(xp_ref, w1_ref, degp_ref, hw1_ref, dinvb_ref, dinv2b_ref,
                 dinv1d_ref):
    hw1_ref[...] = jnp.dot(xp_ref[...], w1_ref[...],
                           preferred_element_type=jnp.float32)
    degb = degp_ref[0] + degp_ref[1] + 1.0
    dinvb = lax.rsqrt(degb)
    dinvb_ref[...] = dinvb
    dinv2b_ref[...] = 1.0 / degb
    dinv1d_ref[...] = lax.rsqrt(jnp.sum(degb, axis=1) * (1.0 / DH))


def _ew_body(cbar_ref, ea_ref, wev_ref, ew_ref):
    blk = ea_ref[...]
    w = wev_ref[...]
    ew_ref[...] = jnp.sum(blk * w, axis=-1) + cbar_ref[0]


def _mid_body(aggp_ref, hw1_ref, dinvb_ref, dinv2b_ref, b1_ref, h_ref):
    agg = aggp_ref[0] + aggp_ref[1]
    h = dinvb_ref[...] * agg + dinv2b_ref[...] * hw1_ref[...] + b1_ref[...]
    h_ref[...] = jnp.maximum(h, 0.0)


def _fin_body(aggp_ref, h_ref, dinvb_ref, dinv2b_ref, w2_ref, b2_ref, out_ref):
    pre = dinvb_ref[...] * (aggp_ref[0] + aggp_ref[1]) \
        + dinv2b_ref[...] * h_ref[...]
    out_ref[...] = jnp.dot(pre, w2_ref[...],
                           preferred_element_type=jnp.float32) + b2_ref[...]


def kernel(x, edge_index, edge_attr, We, be, W1, b1, W2, b2):
    dea = We.shape[0]

    wevec = jnp.mean(We, axis=1)
    cbar = jnp.mean(be).reshape(1)

    degp = _deg_call(edge_index)

    ER = E // 128
    RB = 128
    ea3 = edge_attr.reshape(ER, 128, dea)
    ew2d = pl.pallas_call(
        _ew_body,
        grid=(pl.cdiv(ER, RB),),
        in_specs=[
            pl.BlockSpec(memory_space=pltpu.SMEM),
            pl.BlockSpec((RB, 128, dea), lambda i: (i, 0, 0)),
            pl.BlockSpec((1, 1, dea), lambda i: (0, 0, 0)),
        ],
        out_specs=pl.BlockSpec((RB, 128), lambda i: (i, 0)),
        out_shape=jax.ShapeDtypeStruct((ER, 128), jnp.float32),
    )(cbar, ea3, wevec.reshape(1, 1, dea))
    ew = ew2d.reshape(E)

    hw1, dinvb, dinv2b, dinv1d = pl.pallas_call(
        _dense1_body,
        out_shape=[
            jax.ShapeDtypeStruct((N, DH), jnp.float32),
            jax.ShapeDtypeStruct((N, DH), jnp.float32),
            jax.ShapeDtypeStruct((N, DH), jnp.float32),
            jax.ShapeDtypeStruct((N,), jnp.float32),
        ],
    )(x, W1, degp)

    agg1 = _agg_call(edge_index, ew, dinv1d, hw1)

    h = pl.pallas_call(
        _mid_body,
        out_shape=jax.ShapeDtypeStruct((N, DH), jnp.float32),
    )(agg1, hw1, dinvb, dinv2b, b1.reshape(1, DH))

    agg2 = _agg_call(edge_index, ew, dinv1d, h)

    out = pl.pallas_call(
        _fin_body,
        out_shape=jax.ShapeDtypeStruct((N, DOUT), jnp.float32),
    )(agg2, h, dinvb, dinv2b, W2, b2.reshape(1, DOUT))

    return out


# speedup vs baseline: 40.1466x; 1.1449x over previous
import functools

import jax
import jax.numpy as jnp
from jax import lax
from jax.experimental import pallas as pl
from jax.experimental.pallas import tpu as pltpu
from jax.experimental.pallas import tpu_sc as plsc

N = 10000
NPAD = 10240
E = 320000
DIN = 128
DH = 16
DOUT = 128

NC, NS = 2, 16
NW = NC * NS
CH = E // NW
CHD = E // NS
NBATCH = 5
B = CH // NBATCH
NBATCHD = CHD // B
RPT = NPAD // NS

_mesh = plsc.VectorSubcoreMesh(core_axis_name="c", subcore_axis_name="s")
_sc_params = pltpu.CompilerParams(use_tc_tiling_on_sc=False,
                                  needs_layout_passes=False)


def _rsqrt_nr(x):
    i = plsc.bitcast(x, jnp.int32)
    i = jnp.int32(0x5F3759DF) - lax.shift_right_logical(i, 1)
    y = plsc.bitcast(i, jnp.float32)
    for _ in range(3):
        y = y * (1.5 - 0.5 * x * y * y)
    return y


def _deg_body(ei_hbm, deg_hbm, dinv_hbm, dst_v, ones_v, dinvw_v, deg_sh):
    cid = lax.axis_index("c")
    sid = lax.axis_index("s")
    base = sid * CHD

    pltpu.sync_copy(ei_hbm.at[pl.ds(E + base, CHD)], dst_v)

    def fill_zero(i, _):
        ones_v[i, :] = jnp.zeros((16,), jnp.float32)
        return 0

    lax.fori_loop(0, RPT, fill_zero, 0)
    pltpu.sync_copy(ones_v.at[pl.ds(0, RPT)],
                    deg_sh.at[pl.ds(sid * RPT, RPT)])

    def fill_ones(i, _):
        ones_v[i, :] = jnp.ones((16,), jnp.float32)
        return 0

    lax.fori_loop(0, B, fill_ones, 0)
    plsc.subcore_barrier()

    for k in range(NBATCHD):
        pltpu.sync_copy(ones_v, deg_sh.at[dst_v.at[pl.ds(k * B, B)]], add=True)

    plsc.subcore_barrier()

    pltpu.sync_copy(deg_sh.at[pl.ds(sid * RPT, RPT)],
                    ones_v.at[pl.ds(0, RPT)])

    def dinv16(j, _):
        row16 = j * 16 + lax.iota(jnp.int32, 16)
        lane0 = jnp.zeros((16,), jnp.int32)
        d = plsc.load_gather(ones_v, [row16, lane0]) + 1.0
        dinvw_v[pl.ds(j * 16, 16)] = _rsqrt_nr(d)
        return 0

    lax.fori_loop(0, RPT // 16, dinv16, 0)

    @pl.when(cid == 0)
    def _():
        pltpu.sync_copy(ones_v.at[pl.ds(0, RPT)],
                        deg_hbm.at[pl.ds(sid * RPT, RPT)])

    pltpu.sync_copy(dinvw_v, dinv_hbm.at[cid, pl.ds(sid * RPT, RPT)])


_deg_call = functools.partial(
    pl.kernel,
    out_type=[
        jax.ShapeDtypeStruct((NPAD, DH), jnp.float32),
        jax.ShapeDtypeStruct((NC, NPAD), jnp.float32),
    ],
    mesh=_mesh,
    compiler_params=_sc_params,
    scratch_types=[
        pltpu.VMEM((CHD,), jnp.int32),
        pltpu.VMEM((B, DH), jnp.float32),
        pltpu.VMEM((RPT,), jnp.float32),
        pltpu.VMEM_SHARED((NPAD, DH), jnp.float32),
    ],
)(_deg_body)


def _agg_common(cid, sid, ei_hbm, ew_hbm, feat_hbm, out_hbm, bias_half,
                src_v, dst_v, sc_v, dinv_v, rows_v0, rows_v1, agg_sh,
                gsem, ssem, zero_done):
    base = cid * (E // 2) + sid * CH

    pltpu.sync_copy(ei_hbm.at[pl.ds(base, CH)], src_v)
    pltpu.sync_copy(ei_hbm.at[pl.ds(E + base, CH)], dst_v)
    pltpu.sync_copy(ew_hbm.at[pl.ds(base, CH)], sc_v)

    if not zero_done:
        def fill_zero(i, _):
            rows_v0[i, :] = jnp.zeros((16,), jnp.float32)
            return 0

        lax.fori_loop(0, RPT, fill_zero, 0)
        pltpu.sync_copy(rows_v0.at[pl.ds(0, RPT)],
                        agg_sh.at[pl.ds(sid * RPT, RPT)])

    def scl(j, _):
        s16 = src_v[pl.ds(j * 16, 16)]
        d16 = plsc.load_gather(dinv_v, [s16])
        sc_v[pl.ds(j * 16, 16)] = sc_v[pl.ds(j * 16, 16)] * d16
        if bias_half:
            src_v[pl.ds(j * 16, 16)] = s16 + cid * NPAD
        return 0

    lax.fori_loop(0, CH // 16, scl, 0)
    plsc.subcore_barrier()

    bufs = [rows_v0, rows_v1]
    gd = [None] * NBATCH
    sd = [None] * NBATCH
    gd[0] = pltpu.async_copy(feat_hbm.at[src_v.at[pl.ds(0, B)]], bufs[0], gsem)
    for k in range(NBATCH):
        buf = bufs[k % 2]
        gd[k].wait()
        if k + 1 < NBATCH:
            if k >= 1:
                sd[k - 1].wait()
            gd[k + 1] = pltpu.async_copy(
                feat_hbm.at[src_v.at[pl.ds((k + 1) * B, B)]],
                bufs[(k + 1) % 2], gsem)

        @plsc.parallel_loop(0, B // 16)
        def _(j):
            s16 = sc_v[pl.ds(k * B + j * 16, 16)]
            for i in range(16):
                r = j * 16 + i
                buf[r, :] = buf[r, :] * s16[i]

        sd[k] = pltpu.async_copy(buf, agg_sh.at[dst_v.at[pl.ds(k * B, B)]],
                                 ssem, add=True)
    sd[NBATCH - 2].wait()
    sd[NBATCH - 1].wait()

    plsc.subcore_barrier()
    pltpu.sync_copy(
        agg_sh.at[pl.ds(sid * RPT, RPT)],
        out_hbm.at[cid, pl.ds(sid * RPT, RPT)],
    )


def _agg1_body(ei_hbm, ew_hbm, dinv_hbm, feat_hbm, out_hbm,
               src_v, dst_v, sc_v, dinv_v, rows_v0, rows_v1, agg_sh,
               gsem, ssem):
    cid = lax.axis_index("c")
    sid = lax.axis_index("s")
    pltpu.sync_copy(dinv_hbm.at[cid], dinv_v)
    _agg_common(cid, sid, ei_hbm, ew_hbm, feat_hbm, out_hbm, False,
                src_v, dst_v, sc_v, dinv_v, rows_v0, rows_v1, agg_sh,
                gsem, ssem, zero_done=False)


_agg1_call = functools.partial(
    pl.kernel,
    out_type=jax.ShapeDtypeStruct((NC, NPAD, DH), jnp.float32),
    mesh=_mesh,
    compiler_params=_sc_params,
    scratch_types=[
        pltpu.VMEM((CH,), jnp.int32),
        pltpu.VMEM((CH,), jnp.int32),
        pltpu.VMEM((CH,), jnp.float32),
        pltpu.VMEM((NPAD,), jnp.float32),
        pltpu.VMEM((B, DH), jnp.float32),
        pltpu.VMEM((B, DH), jnp.float32),
        pltpu.VMEM_SHARED((NPAD, DH), jnp.float32),
        pltpu.SemaphoreType.DMA,
        pltpu.SemaphoreType.DMA,
    ],
)(_agg1_body)


def _agg2_body(ei_hbm, ew_hbm, dinv_hbm, a1p_hbm, hw1_hbm, b1_hbm,
               out_hbm, h2_hbm,
               src_v, dst_v, sc_v, dinv_v, rows_v0, rows_v1,
               b1_v, agg_sh, gsem, ssem):
    cid = lax.axis_index("c")
    sid = lax.axis_index("s")
    rbase = sid * RPT

    pltpu.sync_copy(dinv_hbm.at[cid], dinv_v)
    pltpu.sync_copy(b1_hbm, b1_v)

    pltpu.sync_copy(a1p_hbm.at[0, pl.ds(rbase, RPT)],
                    rows_v0.at[pl.ds(0, RPT)])
    pltpu.sync_copy(a1p_hbm.at[1, pl.ds(rbase, RPT)],
                    rows_v0.at[pl.ds(RPT, RPT)])
    pltpu.sync_copy(hw1_hbm.at[pl.ds(rbase, RPT)], rows_v1.at[pl.ds(0, RPT)])
    b1vec = b1_v[...]

    def hrow16(j, _):
        d16 = dinv_v[pl.ds(rbase + j * 16, 16)]
        for i in range(16):
            r = j * 16 + i
            d = d16[i]
            a = rows_v0[r, :] + rows_v0[RPT + r, :]
            h = jnp.maximum(a * d + rows_v1[r, :] * (d * d) + b1vec, 0.0)
            rows_v0[RPT + r, :] = h
        return 0

    lax.fori_loop(0, RPT // 16, hrow16, 0)

    pltpu.sync_copy(rows_v0.at[pl.ds(RPT, RPT)],
                    h2_hbm.at[pl.ds(cid * NPAD + rbase, RPT)])

    def fill_zero(i, _):
        rows_v0[i, :] = jnp.zeros((16,), jnp.float32)
        return 0

    lax.fori_loop(0, RPT, fill_zero, 0)
    pltpu.sync_copy(rows_v0.at[pl.ds(0, RPT)],
                    agg_sh.at[pl.ds(sid * RPT, RPT)])

    _agg_common(cid, sid, ei_hbm, ew_hbm, h2_hbm, out_hbm, True,
                src_v, dst_v, sc_v, dinv_v, rows_v0, rows_v1, agg_sh,
                gsem, ssem, zero_done=True)


_agg2_call = functools.partial(
    pl.kernel,
    out_type=[
        jax.ShapeDtypeStruct((NC, NPAD, DH), jnp.float32),
        jax.ShapeDtypeStruct((NC * NPAD, DH), jnp.float32),
    ],
    mesh=_mesh,
    compiler_params=_sc_params,
    scratch_types=[
        pltpu.VMEM((CH,), jnp.int32),
        pltpu.VMEM((CH,), jnp.int32),
        pltpu.VMEM((CH,), jnp.float32),
        pltpu.VMEM((NPAD,), jnp.float32),
        pltpu.VMEM((B, DH), jnp.float32),
        pltpu.VMEM((B, DH), jnp.float32),
        pltpu.VMEM((DH,), jnp.float32),
        pltpu.VMEM_SHARED((NPAD, DH), jnp.float32),
        pltpu.SemaphoreType.DMA,
        pltpu.SemaphoreType.DMA,
    ],
)(_agg2_body)


def _hw1_body(x_ref, w1_ref, hw1_ref):
    hw1 = jnp.dot(x_ref[...], w1_ref[...], preferred_element_type=jnp.float32)
    hw1_ref[...] = jnp.concatenate(
        [hw1, jnp.zeros((NPAD - N, DH), jnp.float32)], axis=0)


def _ew_body(cbar_ref, ea_ref, wev_ref, ew_ref):
    blk = ea_ref[...]
    w = wev_ref[...]
    ew_ref[...] = jnp.sum(blk * w, axis=-1) + cbar_ref[0]


def _fin_body(aggp_ref, h_ref, deg_ref, w2_ref, b2_ref, out_ref):
    degb = deg_ref[...] + 1.0
    dinvb = lax.rsqrt(degb)
    dinv2b = 1.0 / degb
    pre = dinvb * (aggp_ref[0] + aggp_ref[1]) + dinv2b * h_ref[...]
    out_ref[...] = jnp.dot(pre, w2_ref[...],
                           preferred_element_type=jnp.float32) + b2_ref[...]


def kernel(x, edge_index, edge_attr, We, be, W1, b1, W2, b2):
    dea = We.shape[0]

    wevec = jnp.mean(We, axis=1)
    cbar = jnp.mean(be).reshape(1)
    ei_flat = edge_index.reshape(2 * E)

    hw1 = pl.pallas_call(
        _hw1_body,
        out_shape=jax.ShapeDtypeStruct((NPAD, DH), jnp.float32),
    )(x, W1)

    ER = E // 128
    RB = 128
    ea3 = edge_attr.reshape(ER, 128, dea)
    ew2d = pl.pallas_call(
        _ew_body,
        grid=(pl.cdiv(ER, RB),),
        in_specs=[
            pl.BlockSpec(memory_space=pltpu.SMEM),
            pl.BlockSpec((RB, 128, dea), lambda i: (i, 0, 0)),
            pl.BlockSpec((1, 1, dea), lambda i: (0, 0, 0)),
        ],
        out_specs=pl.BlockSpec((RB, 128), lambda i: (i, 0)),
        out_shape=jax.ShapeDtypeStruct((ER, 128), jnp.float32),
    )(cbar, ea3, wevec.reshape(1, 1, dea))
    ew = ew2d.reshape(E)

    deg, dinv2 = _deg_call(ei_flat)

    agg1 = _agg1_call(ei_flat, ew, dinv2, hw1)

    agg2, h2 = _agg2_call(ei_flat, ew, dinv2, agg1, hw1, b1)

    out = pl.pallas_call(
        _fin_body,
        grid=(1,),
        in_specs=[
            pl.BlockSpec((NC, N, DH), lambda i: (0, 0, 0)),
            pl.BlockSpec((N, DH), lambda i: (0, 0)),
            pl.BlockSpec((N, DH), lambda i: (0, 0)),
            pl.BlockSpec((DH, DOUT), lambda i: (0, 0)),
            pl.BlockSpec((1, DOUT), lambda i: (0, 0)),
        ],
        out_specs=pl.BlockSpec((N, DOUT), lambda i: (0, 0)),
        out_shape=jax.ShapeDtypeStruct((N, DOUT), jnp.float32),
    )(agg2, h2, deg, W2, b2.reshape(1, DOUT))

    return out
